# baseline (speedup 1.0000x reference)
import functools

import jax
import jax.numpy as jnp
from jax import lax
from jax.experimental import pallas as pl
from jax.experimental.pallas import tpu as pltpu
from jax.experimental.pallas import tpu_sc as plsc

NUM_EXPERTS = 64
TOP_K = 8
HIDDEN = 768
N_TOKENS = 32768

_HI16 = -65536
_POS = 0x7FFFFFFF
_NEGFLIP = 0x7FF0000 * 16 + 0xF0000

_info = plsc.get_sparse_core_info()
_NC, _NS = _info.num_cores, _info.num_subcores
_NW = _NC * _NS
_C = N_TOKENS // _NW
_H = 2
_CH = _C // _H
_G = _CH // 32


def _matmul_body(x_ref, w_ref, b_ref, kt_ref):
    r = x_ref.shape[0]
    l32 = jax.lax.dot_general(
        w_ref[...], x_ref[...],
        dimension_numbers=(((1,), (1,)), ((), ())),
        preferred_element_type=jnp.float32,
    ) + b_ref[...]
    bits = jax.lax.bitcast_convert_type(l32, jnp.int32) & _HI16
    mono = bits ^ (jax.lax.shift_right_arithmetic(bits, 31) & _POS)
    eiota = jax.lax.broadcasted_iota(jnp.int32, (NUM_EXPERTS, r), 0)
    kt_ref[...] = (mono & _HI16) | (NUM_EXPERTS - 1 - eiota)


@jax.jit
def _matmul_keys(x, w, b2d):
    n = x.shape[0]
    r = 2048
    return pl.pallas_call(
        _matmul_body,
        grid=(n // r,),
        in_specs=[
            pl.BlockSpec((r, HIDDEN), lambda i: (i, 0)),
            pl.BlockSpec((NUM_EXPERTS, HIDDEN), lambda i: (0, 0)),
            pl.BlockSpec((NUM_EXPERTS, 1), lambda i: (0, 0)),
        ],
        out_specs=pl.BlockSpec((NUM_EXPERTS, r), lambda i: (0, i)),
        out_shape=jax.ShapeDtypeStruct((NUM_EXPERTS, n), jnp.int32),
    )(x, w, b2d)


def _rtne(x16):
    b = jax.lax.bitcast_convert_type(x16, jnp.int32)
    r = b + 32767 + (jax.lax.shift_right_logical(b, 16) & 1)
    return jax.lax.bitcast_convert_type(r & _HI16, jnp.float32)


def _key_val(k):
    vm = k & _HI16
    vbits = jnp.where(k < 0, vm ^ _NEGFLIP, vm)
    return jax.lax.bitcast_convert_type(vbits, jnp.float32)


def _sc_body(kt_hbm, scores_hbm, idx_hbm, key_v, sc_v, scb_v, idx_v, sem):
    wid = lax.axis_index("s") * _NC + lax.axis_index("c")
    lane = lax.iota(jnp.int32, 16)
    int_min = jnp.full((16,), -2147483647 - 1, jnp.int32)

    def _half(h, _):
        base = pl.multiple_of(wid * _C + h * _CH, 512)
        pltpu.sync_copy(kt_hbm.at[:, pl.ds(pl.multiple_of(base, 8), _CH)], key_v)

        def _zero(t, c):
            sc_v[pl.ds(t * 16, 16)] = jnp.zeros((16,), jnp.float32)
            return c

        lax.fori_loop(0, _CH * NUM_EXPERTS // 16, _zero, 0)

        def _group(g, c):
            tops = []
            for sub in range(2):
                off = g * 32 + sub * 16
                ks = [int_min] * TOP_K
                for e in range(NUM_EXPERTS):
                    x = key_v[e, pl.ds(off, 16)]
                    for j in range(TOP_K):
                        cgt = x > ks[j]
                        nk = jnp.where(cgt, x, ks[j])
                        x = jnp.where(cgt, ks[j], x)
                        ks[j] = nk
                tops.append((off, ks))

            for off, ks in tops:
                tok = off + lane
                v = [_key_val(ks[j]) for j in range(TOP_K)]
                ei = [(NUM_EXPERTS - 1) - (ks[j] & (NUM_EXPERTS - 1))
                      for j in range(TOP_K)]
                d = [_rtne(v[j] - v[0]) for j in range(TOP_K)]
                e_ = [_rtne(jnp.exp(d[j])) for j in range(TOP_K)]
                s = e_[0]
                for j in range(1, TOP_K):
                    s = _rtne(s + e_[j])
                for j in range(TOP_K):
                    p = _rtne(e_[j] / s)
                    plsc.store_scatter(sc_v, [tok * NUM_EXPERTS + ei[j]], p)
                    plsc.store_scatter(idx_v, [tok * TOP_K + j], ei[j])
            return c

        lax.fori_loop(0, _G, _group, 0)

        def _tobf16(i, c):
            ia = i * 32 + lane * 2
            a = plsc.load_gather(sc_v, [ia])
            b = plsc.load_gather(sc_v, [ia + 1])
            ab = jax.lax.bitcast_convert_type(a, jnp.int32)
            bb = jax.lax.bitcast_convert_type(b, jnp.int32)
            scb_v[pl.ds(i * 16, 16)] = (
                jax.lax.shift_right_logical(ab, 16) | (bb & _HI16))
            return c

        lax.fori_loop(0, _CH * NUM_EXPERTS // 32, _tobf16, 0)

        pltpu.sync_copy(
            scb_v,
            scores_hbm.at[pl.ds(pl.multiple_of(base * (NUM_EXPERTS // 2), 8), _CH * NUM_EXPERTS // 2)])
        pltpu.sync_copy(idx_v, idx_hbm.at[pl.ds(pl.multiple_of(base * TOP_K, 8), _CH * TOP_K)])
        return _

    lax.fori_loop(0, _H, _half, 0)


_sc_topk = pl.kernel(
    _sc_body,
    out_type=[
        jax.ShapeDtypeStruct((N_TOKENS * NUM_EXPERTS // 2,), jnp.int32),
        jax.ShapeDtypeStruct((N_TOKENS * TOP_K,), jnp.int32),
    ],
    mesh=plsc.VectorSubcoreMesh(core_axis_name="c", subcore_axis_name="s"),
    compiler_params=pltpu.CompilerParams(
        needs_layout_passes=False, use_tc_tiling_on_sc=False),
    scratch_types=[
        pltpu.VMEM((NUM_EXPERTS, _CH), jnp.int32),
        pltpu.VMEM((_CH * NUM_EXPERTS,), jnp.float32),
        pltpu.VMEM((_CH * NUM_EXPERTS // 2,), jnp.int32),
        pltpu.VMEM((_CH * TOP_K,), jnp.int32),
        pltpu.SemaphoreType.DMA,
    ],
)


def kernel(hidden_states, weight, bias):
    x = hidden_states.reshape(-1, HIDDEN)
    b2d = bias.astype(jnp.float32).reshape(NUM_EXPERTS, 1)
    kt = _matmul_keys(x, weight, b2d)
    scores_i32, idx = _sc_topk(kt)
    scores = jax.lax.bitcast_convert_type(scores_i32, jnp.bfloat16)
    return (scores.reshape(N_TOKENS, NUM_EXPERTS), idx.reshape(N_TOKENS, TOP_K))


# speedup vs baseline: 1.2304x; 1.2304x over previous
import functools

import jax
import jax.numpy as jnp
from jax import lax
from jax.experimental import pallas as pl
from jax.experimental.pallas import tpu as pltpu
from jax.experimental.pallas import tpu_sc as plsc

NUM_EXPERTS = 64
TOP_K = 8
HIDDEN = 768
N_TOKENS = 32768

_HI16 = -65536
_POS = 0x7FFFFFFF
_NEGFLIP = 0x7FF0000 * 16 + 0xF0000

_info = plsc.get_sparse_core_info()
_NC, _NS = _info.num_cores, _info.num_subcores
_NW = _NC * _NS
_C = N_TOKENS // _NW
_H = 2
_CH = _C // _H
_G = _CH // 32


def _matmul_body(x_ref, w_ref, b_ref, kt_ref):
    r = x_ref.shape[0]
    l32 = jax.lax.dot_general(
        w_ref[...], x_ref[...],
        dimension_numbers=(((1,), (1,)), ((), ())),
        preferred_element_type=jnp.float32,
    ) + b_ref[...]
    bits = jax.lax.bitcast_convert_type(l32, jnp.int32) & _HI16
    mono = bits ^ (jax.lax.shift_right_arithmetic(bits, 31) & _POS)
    eiota = jax.lax.broadcasted_iota(jnp.int32, (NUM_EXPERTS, r), 0)
    kt_ref[...] = (mono & _HI16) | (NUM_EXPERTS - 1 - eiota)


@jax.jit
def _matmul_keys(x, w, b2d):
    n = x.shape[0]
    r = 2048
    return pl.pallas_call(
        _matmul_body,
        grid=(n // r,),
        in_specs=[
            pl.BlockSpec((r, HIDDEN), lambda i: (i, 0)),
            pl.BlockSpec((NUM_EXPERTS, HIDDEN), lambda i: (0, 0)),
            pl.BlockSpec((NUM_EXPERTS, 1), lambda i: (0, 0)),
        ],
        out_specs=pl.BlockSpec((NUM_EXPERTS, r), lambda i: (0, i)),
        out_shape=jax.ShapeDtypeStruct((NUM_EXPERTS, n), jnp.int32),
    )(x, w, b2d)


def _rtne(x16):
    b = jax.lax.bitcast_convert_type(x16, jnp.int32)
    r = b + 32767 + (jax.lax.shift_right_logical(b, 16) & 1)
    return jax.lax.bitcast_convert_type(r & _HI16, jnp.float32)


def _key_val(k):
    vm = k & _HI16
    vbits = jnp.where(k < 0, vm ^ _NEGFLIP, vm)
    return jax.lax.bitcast_convert_type(vbits, jnp.float32)


def _sc_body(kt_hbm, scores_hbm, idx_hbm, key_v, sc_v, idx_v, sem):
    wid = lax.axis_index("s") * _NC + lax.axis_index("c")
    lane = lax.iota(jnp.int32, 16)
    int_min = jnp.full((16,), -2147483647 - 1, jnp.int32)

    def _half(h, _):
        base = pl.multiple_of(wid * _C + h * _CH, 512)
        pltpu.sync_copy(kt_hbm.at[:, pl.ds(pl.multiple_of(base, 8), _CH)], key_v)

        def _zero(t, c):
            sc_v[pl.ds(t * 16, 16)] = jnp.zeros((16,), jnp.float32)
            return c

        lax.fori_loop(0, _CH * NUM_EXPERTS // 16, _zero, 0)

        def _group(g, c):
            tops = []
            for sub in range(2):
                off = g * 32 + sub * 16
                ks = [int_min] * TOP_K
                for e in range(NUM_EXPERTS):
                    x = key_v[e, pl.ds(off, 16)]
                    for j in range(TOP_K):
                        cgt = x > ks[j]
                        nk = jnp.where(cgt, x, ks[j])
                        x = jnp.where(cgt, ks[j], x)
                        ks[j] = nk
                tops.append((off, ks))

            for off, ks in tops:
                tok = off + lane
                v = [_key_val(ks[j]) for j in range(TOP_K)]
                ei = [(NUM_EXPERTS - 1) - (ks[j] & (NUM_EXPERTS - 1))
                      for j in range(TOP_K)]
                d = [_rtne(v[j] - v[0]) for j in range(TOP_K)]
                e_ = [_rtne(jnp.exp(d[j])) for j in range(TOP_K)]
                s = e_[0]
                for j in range(1, TOP_K):
                    s = _rtne(s + e_[j])
                for j in range(TOP_K):
                    p = _rtne(e_[j] / s)
                    plsc.store_scatter(sc_v, [tok * NUM_EXPERTS + ei[j]], p)
                    plsc.store_scatter(idx_v, [tok * TOP_K + j], ei[j])
            return c

        lax.fori_loop(0, _G, _group, 0)

        pltpu.sync_copy(
            sc_v,
            scores_hbm.at[pl.ds(pl.multiple_of(base * NUM_EXPERTS, 8), _CH * NUM_EXPERTS)])
        pltpu.sync_copy(idx_v, idx_hbm.at[pl.ds(pl.multiple_of(base * TOP_K, 8), _CH * TOP_K)])
        return _

    lax.fori_loop(0, _H, _half, 0)


_sc_topk = pl.kernel(
    _sc_body,
    out_type=[
        jax.ShapeDtypeStruct((N_TOKENS * NUM_EXPERTS,), jnp.float32),
        jax.ShapeDtypeStruct((N_TOKENS * TOP_K,), jnp.int32),
    ],
    mesh=plsc.VectorSubcoreMesh(core_axis_name="c", subcore_axis_name="s"),
    compiler_params=pltpu.CompilerParams(
        needs_layout_passes=False, use_tc_tiling_on_sc=False),
    scratch_types=[
        pltpu.VMEM((NUM_EXPERTS, _CH), jnp.int32),
        pltpu.VMEM((_CH * NUM_EXPERTS,), jnp.float32),
        pltpu.VMEM((_CH * TOP_K,), jnp.int32),
        pltpu.SemaphoreType.DMA,
    ],
)


def kernel(hidden_states, weight, bias):
    x = hidden_states.reshape(-1, HIDDEN)
    b2d = bias.astype(jnp.float32).reshape(NUM_EXPERTS, 1)
    kt = _matmul_keys(x, weight, b2d)
    scores32, idx = _sc_topk(kt)
    return (scores32.reshape(N_TOKENS, NUM_EXPERTS).astype(jnp.bfloat16),
            idx.reshape(N_TOKENS, TOP_K))
